# Initial kernel scaffold; baseline (speedup 1.0000x reference)
import jax
import jax.numpy as jnp
from jax.experimental import pallas as pl


def kernel(x, embed, W1, b1, gamma, beta, W2, b2):
    raise NotImplementedError("write your pallas kernel here")


# speedup vs baseline: 23.4988x; 23.4988x over previous
import functools

import jax
import jax.numpy as jnp
from jax import lax
from jax.experimental import pallas as pl
from jax.experimental.pallas import tpu as pltpu
from jax.experimental.pallas import tpu_sc as plsc

EPS = 1e-5

NC = 2
NS = 16
NW = NC * NS
LANE = 16

CHUNK = 100
G = 2
IDXG = 16


def _pool_kernel(B, SEQ, V, D):
    SPW = B // NW
    NGRP = SPW // G
    JROWS = G * (SEQ // CHUNK)
    NACC = D // LANE
    UNR = 4
    inv_seq = 1.0 / SEQ

    mesh = plsc.VectorSubcoreMesh(
        core_axis_name="c", subcore_axis_name="s",
        num_cores=NC, num_subcores=NS)

    @functools.partial(
        pl.kernel,
        out_type=jax.ShapeDtypeStruct((B, D), jnp.float32),
        mesh=mesh,
        scratch_types=[
            pltpu.VMEM((IDXG * JROWS, CHUNK), jnp.int32),
            pltpu.VMEM((2, G * SEQ, D), jnp.float32),
            pltpu.VMEM((SPW, D), jnp.float32),
            pltpu.SemaphoreType.DMA,
            pltpu.SemaphoreType.DMA,
        ],
        compiler_params=pltpu.CompilerParams(use_tc_tiling_on_sc=False),
    )
    def pool(x2_hbm, embed_hbm, out_hbm, idx_v, rows_v, pooled_v, sem0, sem1):
        wid = lax.axis_index("s") * NC + lax.axis_index("c")
        wbase2 = wid * (SPW * 2)

        sems = (sem0, sem1)

        def gather_copy(buf, g, j):
            irow = (g % IDXG) * JROWS + j
            return pltpu.make_async_copy(
                embed_hbm.at[idx_v.at[irow]],
                rows_v.at[buf, pl.ds(j * CHUNK, CHUNK)],
                sems[buf])

        def issue(buf, g):
            for j in range(JROWS):
                gather_copy(buf, g, j).start()

        def drain(buf, g):
            for j in range(JROWS):
                gather_copy(buf, g, j).wait()

        def refill(nb):
            pltpu.sync_copy(
                x2_hbm.at[pl.ds(wbase2 + nb * (IDXG * JROWS), IDXG * JROWS)],
                idx_v)

        def accum_group(buf, g):
            rows = rows_v.at[buf]
            for s in range(G):
                base = s * SEQ

                def rbody(r, accs):
                    accs = list(accs)
                    for u in range(UNR):
                        row = base + r * UNR + u
                        for c in range(NACC):
                            k = (u % 2) * NACC + c
                            accs[k] = accs[k] + rows[row, pl.ds(c * LANE, LANE)]
                    return tuple(accs)

                zero = jnp.zeros((LANE,), jnp.float32)
                accs = lax.fori_loop(0, SEQ // UNR, rbody, (zero,) * (2 * NACC))
                prow = g * G + s
                for c in range(NACC):
                    pooled_v[prow, pl.ds(c * LANE, LANE)] = (
                        (accs[c] + accs[NACC + c]) * inv_seq)

        refill(0)
        issue(0, 0)

        def outer(i, _):
            g0 = 2 * i
            g1 = g0 + 1
            drain(0, g0)
            issue(1, g1)
            accum_group(0, g0)
            drain(1, g1)

            @pl.when(i < NGRP // 2 - 1)
            def _():
                g2 = g0 + 2

                @pl.when(g2 % IDXG == 0)
                def _():
                    refill(g2 // IDXG)

                issue(0, g2)

            accum_group(1, g1)
            return 0

        lax.fori_loop(0, NGRP // 2, outer, 0)
        pltpu.sync_copy(pooled_v, out_hbm.at[pl.ds(wid * SPW, SPW)])

    return pool


def _stats_call(pooled, W1, b1r, B, D, H, blk):
    nblk = B // blk

    def body(p_ref, w1_ref, b1_ref, muh_ref, varh_ref, sum_scr, s_scr):
        i = pl.program_id(0)

        @pl.when(i == 0)
        def _():
            sum_scr[...] = jnp.zeros_like(sum_scr)
            s_scr[...] = jnp.zeros_like(s_scr)

        p = p_ref[...]
        sum_scr[...] += jnp.sum(p, axis=0, keepdims=True)
        s_scr[...] += lax.dot_general(
            p, p, (((0,), (0,)), ((), ())), preferred_element_type=jnp.float32)

        @pl.when(i == nblk - 1)
        def _():
            mu_p = sum_scr[...] * (1.0 / B)
            cov = s_scr[...] * (1.0 / B) - lax.dot_general(
                mu_p, mu_p, (((0,), (0,)), ((), ())),
                preferred_element_type=jnp.float32)
            w1 = w1_ref[...]
            muh_ref[...] = (
                jnp.dot(mu_p, w1, preferred_element_type=jnp.float32)
                + b1_ref[...])
            m = jnp.dot(cov, w1, preferred_element_type=jnp.float32)
            varh_ref[...] = jnp.sum(w1 * m, axis=0, keepdims=True)

    return pl.pallas_call(
        body,
        grid=(nblk,),
        in_specs=[
            pl.BlockSpec((blk, D), lambda i: (i, 0)),
            pl.BlockSpec((D, H), lambda i: (0, 0)),
            pl.BlockSpec((1, H), lambda i: (0, 0)),
        ],
        out_specs=[
            pl.BlockSpec((1, H), lambda i: (0, 0)),
            pl.BlockSpec((1, H), lambda i: (0, 0)),
        ],
        out_shape=[
            jax.ShapeDtypeStruct((1, H), jnp.float32),
            jax.ShapeDtypeStruct((1, H), jnp.float32),
        ],
        scratch_shapes=[
            pltpu.VMEM((1, D), jnp.float32),
            pltpu.VMEM((D, D), jnp.float32),
        ],
    )(pooled, W1, b1r)


def _apply_call(pooled, W1, b1r, gammar, betar, W2, b2r, muh, varh,
                B, D, H, CLS, blk):
    nblk = B // blk

    def body(p_ref, w1_ref, b1_ref, g_ref, be_ref, w2_ref, b2_ref,
             muh_ref, varh_ref, o_ref):
        h = jnp.dot(p_ref[...], w1_ref[...],
                    preferred_element_type=jnp.float32) + b1_ref[...]
        scale = g_ref[...] * lax.rsqrt(varh_ref[...] + EPS)
        hn = (h - muh_ref[...]) * scale + be_ref[...]
        hr = jnp.maximum(hn, 0.0)
        o_ref[...] = jnp.dot(hr, w2_ref[...],
                             preferred_element_type=jnp.float32) + b2_ref[...]

    full = lambda i: (0, 0)
    return pl.pallas_call(
        body,
        grid=(nblk,),
        in_specs=[
            pl.BlockSpec((blk, D), lambda i: (i, 0)),
            pl.BlockSpec((D, H), full),
            pl.BlockSpec((1, H), full),
            pl.BlockSpec((1, H), full),
            pl.BlockSpec((1, H), full),
            pl.BlockSpec((H, CLS), full),
            pl.BlockSpec((1, CLS), full),
            pl.BlockSpec((1, H), full),
            pl.BlockSpec((1, H), full),
        ],
        out_specs=pl.BlockSpec((blk, CLS), lambda i: (i, 0)),
        out_shape=jax.ShapeDtypeStruct((B, CLS), jnp.float32),
    )(pooled, W1, b1r, gammar, betar, W2, b2r, muh, varh)


def kernel(x, embed, W1, b1, gamma, beta, W2, b2):
    B, SEQ = x.shape
    V, D = embed.shape
    H = W1.shape[1]
    CLS = W2.shape[1]

    x2 = x.astype(jnp.int32).reshape(B * (SEQ // CHUNK), CHUNK)
    pooled = _pool_kernel(B, SEQ, V, D)(x2, embed)

    b1r = b1.reshape(1, H)
    muh, varh = _stats_call(pooled, W1, b1r, B, D, H, blk=1024)
    out = _apply_call(
        pooled, W1, b1r, gamma.reshape(1, H), beta.reshape(1, H),
        W2, b2.reshape(1, CLS), muh, varh, B, D, H, CLS, blk=1024)
    return out
